# baseline (speedup 1.0000x reference)
import functools

import jax
import jax.numpy as jnp
from jax.experimental import pallas as pl
from jax.experimental.pallas import tpu as pltpu

EPS = 1e-06
_NFINE = 32
_TROWS = 40


def _stats_kernel(x_ref, t_ref, w_ref, o_ref):
    x = x_ref[...]
    t = t_ref[...]
    w = w_ref[...] + EPS
    l1 = jnp.abs(x - t)
    s = jnp.sum(l1 * w + l1 / w)
    lane = jax.lax.broadcasted_iota(jnp.int32, (1, 1, 128), 2)
    vals = jnp.where(lane == 0, s,
           jnp.where(lane == 1, jnp.min(x),
           jnp.where(lane == 2, jnp.max(x),
           jnp.where(lane == 3, jnp.min(t), jnp.max(t)))))
    o_ref[...] = vals


def _hist_kernel(p_ref, x_ref, t_ref, hp_ref, hg_ref, *, rows, w, ncoarse):
    h = pl.program_id(1)

    @pl.when(h == 0)
    def _():
        hp_ref[...] = jnp.zeros_like(hp_ref)
        hg_ref[...] = jnp.zeros_like(hg_ref)

    bf = jnp.bfloat16
    q_iota = jax.lax.broadcasted_iota(
        jnp.int32, (_TROWS, w), 0).astype(bf)
    sub_iota = jax.lax.broadcasted_iota(
        jnp.int32, (ncoarse, w), 0).astype(bf)
    one = jnp.ones((), bf)
    zero = jnp.zeros((), bf)

    def accum(src_ref, mn, scale, out_ref):
        def body(i, acc):
            x8 = src_ref[0, pl.ds(i * 64, 64), :]
            tt = (x8 - mn) * scale
            r = jnp.floor(tt * (1.0 / _NFINE))
            v8 = (tt - _NFINE * r).astype(bf)
            r8 = r.astype(bf)
            for j in range(64):
                v = v8[j:j + 1, :]
                rj = r8[j:j + 1, :]
                lhs = one - jnp.abs(jax.lax.clamp(-one, v - q_iota, one))
                rhs = jnp.where(rj == sub_iota, one, zero)
                acc = acc + jax.lax.dot_general(
                    lhs.astype(jnp.float8_e4m3fn), rhs.astype(jnp.float8_e4m3fn),
                    (((1,), (1,)), ((), ())),
                    preferred_element_type=jnp.float32)
            return acc
        acc = jax.lax.fori_loop(
            0, rows // 64, body, jnp.zeros((_TROWS, ncoarse), jnp.float32))
        out_ref[0] += acc

    accum(x_ref, p_ref[0], p_ref[1], hp_ref)
    accum(t_ref, p_ref[2], p_ref[3], hg_ref)


def _final_kernel(ps_ref, hp_ref, hg_ref, w2_ref, o_ref, *, b, ncoarse):
    def fixup(h24):
        h16 = h24[:, 0:_NFINE, :]
        carry = h24[:, _NFINE:_NFINE + 1, :]
        z = jnp.zeros((b, 1, 1), jnp.float32)
        shifted = jnp.concatenate([z, carry[:, :, 0:ncoarse - 1]], axis=2)
        row0 = h16[:, 0:1, :] + shifted
        h16 = jnp.concatenate([row0, h16[:, 1:_NFINE, :]], axis=1)
        iq = jax.lax.broadcasted_iota(jnp.int32, (b, _NFINE, ncoarse), 1)
        ir = jax.lax.broadcasted_iota(jnp.int32, (b, _NFINE, ncoarse), 2)
        drop = ((iq == 0) & (ir == 0)) | ((iq == _NFINE - 1) & (ir == ncoarse - 1))
        return jnp.where(drop, 0.0, h16)

    hp = fixup(hp_ref[...])
    hg = fixup(hg_ref[...])
    p = hp / jnp.sum(hp)
    g = hg / jnp.sum(hg)
    kld = jnp.abs(jnp.exp(g) * (g - p))
    w2 = w2_ref[...] + EPS
    pb = jnp.sum(kld * w2 + kld / w2) / (b * _NFINE * ncoarse)
    o_ref[...] = jnp.full((1, 128), 4.0 * ps_ref[0] + pb)


def kernel(inputo, target, we1, we2):
    b, c, hh, w = inputo.shape
    bins = w
    ncoarse = bins // _NFINE
    rows_tot = b * c * hh

    ra = min(512, rows_tot)
    ga = rows_tot // ra
    x2 = inputo.reshape(rows_tot, w)
    t2 = target.reshape(rows_tot, w)
    w1 = we1.reshape(rows_tot, w)

    stats = pl.pallas_call(
        _stats_kernel,
        out_shape=jax.ShapeDtypeStruct((ga, 1, 128), jnp.float32),
        grid=(ga,),
        in_specs=[pl.BlockSpec((ra, w), lambda i: (i, 0)) for _ in range(3)],
        out_specs=pl.BlockSpec((1, 1, 128), lambda i: (i, 0, 0)),
        compiler_params=pltpu.CompilerParams(
            dimension_semantics=("parallel",),
            vmem_limit_bytes=48 * 1024 * 1024,
        ),
        name="loss_stats",
    )(x2, t2, w1)

    n_elem = b * c * hh * w
    parta = jnp.sum(stats[:, 0, 0]) / n_elem
    mn_x = jnp.min(stats[:, 0, 1])
    mx_x = jnp.max(stats[:, 0, 2])
    mn_t = jnp.min(stats[:, 0, 3])
    mx_t = jnp.max(stats[:, 0, 4])
    params = jnp.stack([mn_x, bins / (mx_x - mn_x), mn_t, bins / (mx_t - mn_t)])

    rb = min(512, c * hh)
    hc = (c * hh) // rb
    x3 = inputo.reshape(b, c * hh, w)
    t3 = target.reshape(b, c * hh, w)

    hist_fn = functools.partial(_hist_kernel, rows=rb, w=w, ncoarse=ncoarse)
    hp24, hg24 = pl.pallas_call(
        hist_fn,
        out_shape=[jax.ShapeDtypeStruct((b, _TROWS, ncoarse), jnp.float32)] * 2,
        grid=(b, hc),
        in_specs=[
            pl.BlockSpec(memory_space=pltpu.SMEM),
            pl.BlockSpec((1, rb, w), lambda i, j: (i, j, 0)),
            pl.BlockSpec((1, rb, w), lambda i, j: (i, j, 0)),
        ],
        out_specs=[
            pl.BlockSpec((1, _TROWS, ncoarse), lambda i, j: (i, 0, 0))
            for _ in range(2)
        ],
        compiler_params=pltpu.CompilerParams(
            dimension_semantics=("parallel", "arbitrary"),
            vmem_limit_bytes=48 * 1024 * 1024,
        ),
        name="loss_hist",
    )(params, x3, t3)

    we2r = we2.reshape(b, bins).reshape(b, ncoarse, _NFINE).transpose(0, 2, 1)
    final_fn = functools.partial(_final_kernel, b=b, ncoarse=ncoarse)
    pvec = pl.pallas_call(
        final_fn,
        out_shape=jax.ShapeDtypeStruct((1, 128), jnp.float32),
        grid=(1,),
        in_specs=[
            pl.BlockSpec(memory_space=pltpu.SMEM),
            pl.BlockSpec((b, _TROWS, ncoarse), lambda i: (0, 0, 0)),
            pl.BlockSpec((b, _TROWS, ncoarse), lambda i: (0, 0, 0)),
            pl.BlockSpec((b, _NFINE, ncoarse), lambda i: (0, 0, 0)),
        ],
        out_specs=pl.BlockSpec((1, 128), lambda i: (0, 0)),
        name="loss_final",
    )(jnp.stack([parta]), hp24, hg24, we2r)
    return pvec[0, 0]


# speedup vs baseline: 1.0186x; 1.0186x over previous
import functools

import jax
import jax.numpy as jnp
from jax.experimental import pallas as pl
from jax.experimental.pallas import tpu as pltpu

EPS = 1e-06
_NFINE = 32
_TROWS = 40


def _stats_kernel(x_ref, t_ref, w_ref, o_ref):
    x = x_ref[...]
    t = t_ref[...]
    w = w_ref[...] + EPS
    l1 = jnp.abs(x - t)
    s = jnp.sum(l1 * w + l1 / w)
    lane = jax.lax.broadcasted_iota(jnp.int32, (1, 1, 128), 2)
    vals = jnp.where(lane == 0, s,
           jnp.where(lane == 1, jnp.min(x),
           jnp.where(lane == 2, jnp.max(x),
           jnp.where(lane == 3, jnp.min(t), jnp.max(t)))))
    o_ref[...] = vals


def _hist_kernel(p_ref, x_ref, t_ref, hp_ref, hg_ref, *, rows, w, ncoarse):
    h = pl.program_id(1)

    @pl.when(h == 0)
    def _():
        hp_ref[...] = jnp.zeros_like(hp_ref)
        hg_ref[...] = jnp.zeros_like(hg_ref)

    bf = jnp.bfloat16
    q_iota = jax.lax.broadcasted_iota(
        jnp.int32, (_TROWS, w), 0).astype(bf)
    sub_iota = jax.lax.broadcasted_iota(
        jnp.int32, (ncoarse, w), 0).astype(bf)
    one = jnp.ones((), bf)
    zero = jnp.zeros((), bf)

    def accum(src_ref, mn, scale, out_ref):
        def body(i, acc):
            x8 = src_ref[0, pl.ds(i * 128, 128), :]
            tt = (x8 - mn) * scale
            r = jnp.floor(tt * (1.0 / _NFINE))
            v8 = (tt - _NFINE * r).astype(bf)
            r8 = r.astype(bf)
            for j in range(128):
                v = v8[j:j + 1, :]
                rj = r8[j:j + 1, :]
                lhs = jnp.maximum(one - jnp.abs(v - q_iota), zero)
                rhs = jnp.where(rj == sub_iota, one, zero)
                acc = acc + jax.lax.dot_general(
                    lhs.astype(jnp.float8_e4m3fn), rhs.astype(jnp.float8_e4m3fn),
                    (((1,), (1,)), ((), ())),
                    preferred_element_type=jnp.float32)
            return acc
        acc = jax.lax.fori_loop(
            0, rows // 128, body, jnp.zeros((_TROWS, ncoarse), jnp.float32))
        out_ref[0] += acc

    accum(x_ref, p_ref[0], p_ref[1], hp_ref)
    accum(t_ref, p_ref[2], p_ref[3], hg_ref)


def _final_kernel(ps_ref, hp_ref, hg_ref, w2_ref, o_ref, *, b, ncoarse):
    def fixup(h24):
        h16 = h24[:, 0:_NFINE, :]
        carry = h24[:, _NFINE:_NFINE + 1, :]
        z = jnp.zeros((b, 1, 1), jnp.float32)
        shifted = jnp.concatenate([z, carry[:, :, 0:ncoarse - 1]], axis=2)
        row0 = h16[:, 0:1, :] + shifted
        h16 = jnp.concatenate([row0, h16[:, 1:_NFINE, :]], axis=1)
        iq = jax.lax.broadcasted_iota(jnp.int32, (b, _NFINE, ncoarse), 1)
        ir = jax.lax.broadcasted_iota(jnp.int32, (b, _NFINE, ncoarse), 2)
        drop = ((iq == 0) & (ir == 0)) | ((iq == _NFINE - 1) & (ir == ncoarse - 1))
        return jnp.where(drop, 0.0, h16)

    hp = fixup(hp_ref[...])
    hg = fixup(hg_ref[...])
    p = hp / jnp.sum(hp)
    g = hg / jnp.sum(hg)
    kld = jnp.abs(jnp.exp(g) * (g - p))
    w2 = w2_ref[...] + EPS
    pb = jnp.sum(kld * w2 + kld / w2) / (b * _NFINE * ncoarse)
    o_ref[...] = jnp.full((1, 128), 4.0 * ps_ref[0] + pb)


def kernel(inputo, target, we1, we2):
    b, c, hh, w = inputo.shape
    bins = w
    ncoarse = bins // _NFINE
    rows_tot = b * c * hh

    ra = min(512, rows_tot)
    ga = rows_tot // ra
    x2 = inputo.reshape(rows_tot, w)
    t2 = target.reshape(rows_tot, w)
    w1 = we1.reshape(rows_tot, w)

    stats = pl.pallas_call(
        _stats_kernel,
        out_shape=jax.ShapeDtypeStruct((ga, 1, 128), jnp.float32),
        grid=(ga,),
        in_specs=[pl.BlockSpec((ra, w), lambda i: (i, 0)) for _ in range(3)],
        out_specs=pl.BlockSpec((1, 1, 128), lambda i: (i, 0, 0)),
        compiler_params=pltpu.CompilerParams(
            dimension_semantics=("parallel",),
            vmem_limit_bytes=48 * 1024 * 1024,
        ),
        name="loss_stats",
    )(x2, t2, w1)

    n_elem = b * c * hh * w
    parta = jnp.sum(stats[:, 0, 0]) / n_elem
    mn_x = jnp.min(stats[:, 0, 1])
    mx_x = jnp.max(stats[:, 0, 2])
    mn_t = jnp.min(stats[:, 0, 3])
    mx_t = jnp.max(stats[:, 0, 4])
    params = jnp.stack([mn_x, bins / (mx_x - mn_x), mn_t, bins / (mx_t - mn_t)])

    rb = min(512, c * hh)
    hc = (c * hh) // rb
    x3 = inputo.reshape(b, c * hh, w)
    t3 = target.reshape(b, c * hh, w)

    hist_fn = functools.partial(_hist_kernel, rows=rb, w=w, ncoarse=ncoarse)
    hp24, hg24 = pl.pallas_call(
        hist_fn,
        out_shape=[jax.ShapeDtypeStruct((b, _TROWS, ncoarse), jnp.float32)] * 2,
        grid=(b, hc),
        in_specs=[
            pl.BlockSpec(memory_space=pltpu.SMEM),
            pl.BlockSpec((1, rb, w), lambda i, j: (i, j, 0)),
            pl.BlockSpec((1, rb, w), lambda i, j: (i, j, 0)),
        ],
        out_specs=[
            pl.BlockSpec((1, _TROWS, ncoarse), lambda i, j: (i, 0, 0))
            for _ in range(2)
        ],
        compiler_params=pltpu.CompilerParams(
            dimension_semantics=("parallel", "arbitrary"),
            vmem_limit_bytes=48 * 1024 * 1024,
        ),
        name="loss_hist",
    )(params, x3, t3)

    we2r = we2.reshape(b, bins).reshape(b, ncoarse, _NFINE).transpose(0, 2, 1)
    final_fn = functools.partial(_final_kernel, b=b, ncoarse=ncoarse)
    pvec = pl.pallas_call(
        final_fn,
        out_shape=jax.ShapeDtypeStruct((1, 128), jnp.float32),
        grid=(1,),
        in_specs=[
            pl.BlockSpec(memory_space=pltpu.SMEM),
            pl.BlockSpec((b, _TROWS, ncoarse), lambda i: (0, 0, 0)),
            pl.BlockSpec((b, _TROWS, ncoarse), lambda i: (0, 0, 0)),
            pl.BlockSpec((b, _NFINE, ncoarse), lambda i: (0, 0, 0)),
        ],
        out_specs=pl.BlockSpec((1, 128), lambda i: (0, 0)),
        name="loss_final",
    )(jnp.stack([parta]), hp24, hg24, we2r)
    return pvec[0, 0]


# speedup vs baseline: 1.0187x; 1.0001x over previous
import functools

import jax
import jax.numpy as jnp
from jax.experimental import pallas as pl
from jax.experimental.pallas import tpu as pltpu

EPS = 1e-06
_NFINE = 32
_TROWS = 40


def _stats_kernel(x_ref, t_ref, w_ref, o_ref):
    x = x_ref[...]
    t = t_ref[...]
    w = w_ref[...] + EPS
    l1 = jnp.abs(x - t)
    s = jnp.sum(l1 * w + l1 / w)
    lane = jax.lax.broadcasted_iota(jnp.int32, (1, 1, 128), 2)
    vals = jnp.where(lane == 0, s,
           jnp.where(lane == 1, jnp.min(x),
           jnp.where(lane == 2, jnp.max(x),
           jnp.where(lane == 3, jnp.min(t), jnp.max(t)))))
    o_ref[...] = vals


def _hist_kernel(p_ref, x_ref, t_ref, hp_ref, hg_ref, *, rows, w, ncoarse):
    h = pl.program_id(1)

    @pl.when(h == 0)
    def _():
        hp_ref[...] = jnp.zeros_like(hp_ref)
        hg_ref[...] = jnp.zeros_like(hg_ref)

    bf = jnp.bfloat16
    q_iota = jax.lax.broadcasted_iota(
        jnp.int32, (_TROWS, w), 0).astype(bf)
    sub_iota = jax.lax.broadcasted_iota(
        jnp.int32, (ncoarse, w), 0).astype(bf)
    one = jnp.ones((), bf)
    zero = jnp.zeros((), bf)

    un = min(128, rows)

    def accum(src_ref, mn, scale, out_ref):
        def body(i, acc):
            x8 = src_ref[0, pl.ds(i * un, un), :]
            tt = (x8 - mn) * scale
            r = jnp.floor(tt * (1.0 / _NFINE))
            v8 = (tt - _NFINE * r).astype(bf)
            r8 = r.astype(bf)
            for j in range(un):
                v = v8[j:j + 1, :]
                rj = r8[j:j + 1, :]
                lhs = jnp.maximum(one - jnp.abs(v - q_iota), zero)
                rhs = jnp.where(rj == sub_iota, one, zero)
                acc = acc + jax.lax.dot_general(
                    lhs.astype(jnp.float8_e4m3fn), rhs.astype(jnp.float8_e4m3fn),
                    (((1,), (1,)), ((), ())),
                    preferred_element_type=jnp.float32)
            return acc
        acc = jax.lax.fori_loop(
            0, rows // un, body, jnp.zeros((_TROWS, ncoarse), jnp.float32))
        out_ref[0] += acc

    accum(x_ref, p_ref[0], p_ref[1], hp_ref)
    accum(t_ref, p_ref[2], p_ref[3], hg_ref)


def _final_kernel(ps_ref, hp_ref, hg_ref, w2_ref, o_ref, *, b, ncoarse):
    def fixup(h24):
        h16 = h24[:, 0:_NFINE, :]
        carry = h24[:, _NFINE:_NFINE + 1, :]
        z = jnp.zeros((b, 1, 1), jnp.float32)
        shifted = jnp.concatenate([z, carry[:, :, 0:ncoarse - 1]], axis=2)
        row0 = h16[:, 0:1, :] + shifted
        h16 = jnp.concatenate([row0, h16[:, 1:_NFINE, :]], axis=1)
        iq = jax.lax.broadcasted_iota(jnp.int32, (b, _NFINE, ncoarse), 1)
        ir = jax.lax.broadcasted_iota(jnp.int32, (b, _NFINE, ncoarse), 2)
        drop = ((iq == 0) & (ir == 0)) | ((iq == _NFINE - 1) & (ir == ncoarse - 1))
        return jnp.where(drop, 0.0, h16)

    hp = fixup(hp_ref[...])
    hg = fixup(hg_ref[...])
    p = hp / jnp.sum(hp)
    g = hg / jnp.sum(hg)
    kld = jnp.abs(jnp.exp(g) * (g - p))
    w2 = w2_ref[...] + EPS
    pb = jnp.sum(kld * w2 + kld / w2) / (b * _NFINE * ncoarse)
    o_ref[...] = jnp.full((1, 128), 4.0 * ps_ref[0] + pb)


def kernel(inputo, target, we1, we2):
    b, c, hh, w = inputo.shape
    bins = w
    ncoarse = bins // _NFINE
    rows_tot = b * c * hh

    ra = min(512, rows_tot)
    ga = rows_tot // ra
    x2 = inputo.reshape(rows_tot, w)
    t2 = target.reshape(rows_tot, w)
    w1 = we1.reshape(rows_tot, w)

    stats = pl.pallas_call(
        _stats_kernel,
        out_shape=jax.ShapeDtypeStruct((ga, 1, 128), jnp.float32),
        grid=(ga,),
        in_specs=[pl.BlockSpec((ra, w), lambda i: (i, 0)) for _ in range(3)],
        out_specs=pl.BlockSpec((1, 1, 128), lambda i: (i, 0, 0)),
        compiler_params=pltpu.CompilerParams(
            dimension_semantics=("parallel",),
            vmem_limit_bytes=48 * 1024 * 1024,
        ),
        name="loss_stats",
    )(x2, t2, w1)

    n_elem = b * c * hh * w
    parta = jnp.sum(stats[:, 0, 0]) / n_elem
    mn_x = jnp.min(stats[:, 0, 1])
    mx_x = jnp.max(stats[:, 0, 2])
    mn_t = jnp.min(stats[:, 0, 3])
    mx_t = jnp.max(stats[:, 0, 4])
    params = jnp.stack([mn_x, bins / (mx_x - mn_x), mn_t, bins / (mx_t - mn_t)])

    rb = min(512, c * hh)
    hc = (c * hh) // rb
    x3 = inputo.reshape(b, c * hh, w)
    t3 = target.reshape(b, c * hh, w)

    hist_fn = functools.partial(_hist_kernel, rows=rb, w=w, ncoarse=ncoarse)
    hp24, hg24 = pl.pallas_call(
        hist_fn,
        out_shape=[jax.ShapeDtypeStruct((b, _TROWS, ncoarse), jnp.float32)] * 2,
        grid=(b, hc),
        in_specs=[
            pl.BlockSpec(memory_space=pltpu.SMEM),
            pl.BlockSpec((1, rb, w), lambda i, j: (i, j, 0)),
            pl.BlockSpec((1, rb, w), lambda i, j: (i, j, 0)),
        ],
        out_specs=[
            pl.BlockSpec((1, _TROWS, ncoarse), lambda i, j: (i, 0, 0))
            for _ in range(2)
        ],
        compiler_params=pltpu.CompilerParams(
            dimension_semantics=("parallel", "arbitrary"),
            vmem_limit_bytes=48 * 1024 * 1024,
        ),
        name="loss_hist",
    )(params, x3, t3)

    we2r = we2.reshape(b, bins).reshape(b, ncoarse, _NFINE).transpose(0, 2, 1)
    final_fn = functools.partial(_final_kernel, b=b, ncoarse=ncoarse)
    pvec = pl.pallas_call(
        final_fn,
        out_shape=jax.ShapeDtypeStruct((1, 128), jnp.float32),
        grid=(1,),
        in_specs=[
            pl.BlockSpec(memory_space=pltpu.SMEM),
            pl.BlockSpec((b, _TROWS, ncoarse), lambda i: (0, 0, 0)),
            pl.BlockSpec((b, _TROWS, ncoarse), lambda i: (0, 0, 0)),
            pl.BlockSpec((b, _NFINE, ncoarse), lambda i: (0, 0, 0)),
        ],
        out_specs=pl.BlockSpec((1, 128), lambda i: (0, 0)),
        name="loss_final",
    )(jnp.stack([parta]), hp24, hg24, we2r)
    return pvec[0, 0]
